# baseline (speedup 1.0000x reference)
import functools

import jax
import jax.numpy as jnp
from jax.experimental import pallas as pl
from jax.experimental.pallas import tpu as pltpu

D_OUT = 2048
D_IN = 2048
BAND = 512
SUP = 512
NSUP = D_IN // SUP
NPLANE = 4
SRC = D_IN // 2


def _band_kernel(xb, ib, c0p, c1p, p0p, p1p, out_ref, h_ref):
    band = pl.program_id(0)
    r0 = band * BAND

    @pl.when(band == 0)
    def _build_h():
        sl = jax.lax.broadcasted_iota(jnp.int32, (SUP // 2, 1), 0)
        for s in range(NSUP):
            for p in range(NPLANE):
                tgt = s * SUP + 4 * (sl >> 1) + p
                acc = jnp.zeros((SUP // 2, SUP), dtype=jnp.float32)
                for j in range(NPLANE):
                    pm = p1p[j:j + 1, s * SUP:(s + 1) * SUP]
                    cm = c1p[j:j + 1, s * SUP:(s + 1) * SUP]
                    acc = acc + jnp.where(pm == tgt, cm, 0.0)
                h_ref[p, s] = acc

    rowid = jax.lax.broadcasted_iota(jnp.int32, (BAND, SRC), 0)
    sid = jax.lax.broadcasted_iota(jnp.int32, (BAND, SRC), 1)
    base = (r0 + rowid) * D_IN + 4 * (sid >> 1)
    off = ib[...] - base
    xv = xb[...]

    parts = []
    for s in range(NSUP):
        acc = None
        for p in range(NPLANE):
            u = jnp.where(off[:, s * 256:(s + 1) * 256] == p,
                          xv[:, s * 256:(s + 1) * 256], 0.0)
            d = jnp.dot(u, h_ref[p, s], preferred_element_type=jnp.float32)
            acc = d if acc is None else acc + d
        parts.append(acc)
    v = jnp.concatenate(parts, axis=1)

    jj = jax.lax.broadcasted_iota(jnp.int32, (BAND, 1), 0)
    nt = jnp.zeros((BAND, BAND), dtype=jnp.float32)
    for j in range(NPLANE):
        po = p0p[j:j + 1, :]
        co = c0p[j:j + 1, :]
        nt = nt + jnp.where(po == r0 + jj, co, 0.0)
    out_ref[...] = jax.lax.dot_general(
        nt, v, (((0,), (0,)), ((), ())), preferred_element_type=jnp.float32)


@functools.partial(jax.jit, static_argnames=("interpret",))
def kernel(X, c_0, c_1, mask_idx, perm0, perm1, interpret=False):
    xb = X.reshape(D_OUT, SRC)
    ib = mask_idx.reshape(D_OUT, SRC)
    c0p = jnp.concatenate([c_0, jnp.zeros((4, D_OUT), jnp.float32)], axis=0)
    c1p = jnp.concatenate([c_1, jnp.zeros((4, D_IN), jnp.float32)], axis=0)
    p0p = jnp.concatenate([perm0, perm0], axis=0)
    p1p = jnp.concatenate([perm1, perm1], axis=0)

    return pl.pallas_call(
        _band_kernel,
        grid=(D_OUT // BAND,),
        in_specs=[
            pl.BlockSpec((BAND, SRC), lambda i: (i, 0)),
            pl.BlockSpec((BAND, SRC), lambda i: (i, 0)),
            pl.BlockSpec((8, BAND), lambda i: (0, i)),
            pl.BlockSpec((8, D_IN), lambda i: (0, 0)),
            pl.BlockSpec((8, BAND), lambda i: (0, i)),
            pl.BlockSpec((8, D_IN), lambda i: (0, 0)),
        ],
        out_specs=pl.BlockSpec((BAND, D_IN), lambda i: (i, 0)),
        out_shape=jax.ShapeDtypeStruct((D_OUT, D_IN), jnp.float32),
        scratch_shapes=[
            pltpu.VMEM((NPLANE, NSUP, SUP // 2, SUP), jnp.float32)],
        interpret=interpret,
    )(xb, ib, c0p, c1p, p0p, p1p)


# speedup vs baseline: 1.1429x; 1.1429x over previous
import functools

import jax
import jax.numpy as jnp
from jax.experimental import pallas as pl
from jax.experimental.pallas import tpu as pltpu

D_OUT = 2048
D_IN = 2048
BAND = 256
SUP = 512
NSUP = D_IN // SUP
NPLANE = 4
SRC = D_IN // 2


def _band_kernel(xb, ib, c0p, c1p, p0p, p1p, out_ref, h_ref):
    band = pl.program_id(0)
    r0 = band * BAND

    @pl.when(band == 0)
    def _build_h():
        sl = jax.lax.broadcasted_iota(jnp.int32, (SUP // 2, 1), 0)
        for s in range(NSUP):
            for p in range(NPLANE):
                tgt = s * SUP + 4 * (sl >> 1) + p
                acc = jnp.zeros((SUP // 2, SUP), dtype=jnp.float32)
                for j in range(NPLANE):
                    pm = p1p[0, j:j + 1, s * SUP:(s + 1) * SUP]
                    cm = c1p[0, j:j + 1, s * SUP:(s + 1) * SUP]
                    acc = acc + jnp.where(pm == tgt, cm, 0.0)
                h_ref[p, s] = acc

    rowid = jax.lax.broadcasted_iota(jnp.int32, (BAND, SRC), 0)
    sid = jax.lax.broadcasted_iota(jnp.int32, (BAND, SRC), 1)
    base = (r0 + rowid) * D_IN + 4 * (sid >> 1)
    off = ib[...] - base
    xv = xb[...]

    parts = []
    for s in range(NSUP):
        acc = None
        for p in range(NPLANE):
            u = jnp.where(off[:, s * 256:(s + 1) * 256] == p,
                          xv[:, s * 256:(s + 1) * 256], 0.0)
            d = jnp.dot(u, h_ref[p, s], preferred_element_type=jnp.float32)
            acc = d if acc is None else acc + d
        parts.append(acc)
    v = jnp.concatenate(parts, axis=1)

    jj = jax.lax.broadcasted_iota(jnp.int32, (BAND, 1), 0)
    nt = jnp.zeros((BAND, BAND), dtype=jnp.float32)
    for j in range(NPLANE):
        po = p0p[0, j:j + 1, :]
        co = c0p[0, j:j + 1, :]
        nt = nt + jnp.where(po == r0 + jj, co, 0.0)
    out_ref[...] = jax.lax.dot_general(
        nt, v, (((0,), (0,)), ((), ())), preferred_element_type=jnp.float32)


@functools.partial(jax.jit, static_argnames=("interpret",))
def kernel(X, c_0, c_1, mask_idx, perm0, perm1, interpret=False):
    xb = X.reshape(D_OUT, SRC)
    ib = mask_idx.reshape(D_OUT, SRC)
    c0p = c_0.reshape(1, 4, D_OUT)
    c1p = c_1.reshape(1, 4, D_IN)
    p0p = perm0.reshape(1, 4, D_OUT)
    p1p = perm1.reshape(1, 4, D_IN)

    return pl.pallas_call(
        _band_kernel,
        grid=(D_OUT // BAND,),
        in_specs=[
            pl.BlockSpec((BAND, SRC), lambda i: (i, 0)),
            pl.BlockSpec((BAND, SRC), lambda i: (i, 0)),
            pl.BlockSpec((1, 4, BAND), lambda i: (0, 0, i)),
            pl.BlockSpec((1, 4, D_IN), lambda i: (0, 0, 0)),
            pl.BlockSpec((1, 4, BAND), lambda i: (0, 0, i)),
            pl.BlockSpec((1, 4, D_IN), lambda i: (0, 0, 0)),
        ],
        out_specs=pl.BlockSpec((BAND, D_IN), lambda i: (i, 0)),
        out_shape=jax.ShapeDtypeStruct((D_OUT, D_IN), jnp.float32),
        scratch_shapes=[
            pltpu.VMEM((NPLANE, NSUP, SUP // 2, SUP), jnp.float32)],
        interpret=interpret,
    )(xb, ib, c0p, c1p, p0p, p1p)
